# baseline (speedup 1.0000x reference)
import jax
import jax.numpy as jnp
from jax.experimental import pallas as pl

BATCH = 4096
MAXLEN = 200
EMBED_DIM = 64

BB = 128


def _add_kernel(x_ref, pos_ref, o_ref):
    o_ref[...] = x_ref[...] + pos_ref[...][None, :, :]


def kernel(x, pos_table):
    return pl.pallas_call(
        _add_kernel,
        grid=(BATCH // BB,),
        in_specs=[
            pl.BlockSpec((BB, MAXLEN, EMBED_DIM), lambda i: (i, 0, 0)),
            pl.BlockSpec((MAXLEN, EMBED_DIM), lambda i: (0, 0)),
        ],
        out_specs=pl.BlockSpec((BB, MAXLEN, EMBED_DIM), lambda i: (i, 0, 0)),
        out_shape=jax.ShapeDtypeStruct((BATCH, MAXLEN, EMBED_DIM), jnp.float32),
    )(x, pos_table)


# speedup vs baseline: 1.3662x; 1.3662x over previous
import jax
import jax.numpy as jnp
from jax.experimental import pallas as pl

BATCH = 4096
MAXLEN = 200
EMBED_DIM = 64

KB = 32
ROWS = KB * MAXLEN


def _add_kernel(x_ref, pos_ref, o_ref):
    v = x_ref[...].reshape(KB, MAXLEN, EMBED_DIM) + pos_ref[...][None]
    o_ref[...] = v.reshape(ROWS, EMBED_DIM)


def kernel(x, pos_table):
    x2 = x.reshape(BATCH * MAXLEN, EMBED_DIM)
    out = pl.pallas_call(
        _add_kernel,
        grid=(BATCH // KB,),
        in_specs=[
            pl.BlockSpec((ROWS, EMBED_DIM), lambda i: (i, 0)),
            pl.BlockSpec((MAXLEN, EMBED_DIM), lambda i: (0, 0)),
        ],
        out_specs=pl.BlockSpec((ROWS, EMBED_DIM), lambda i: (i, 0)),
        out_shape=jax.ShapeDtypeStruct((BATCH * MAXLEN, EMBED_DIM), jnp.float32),
    )(x2, pos_table)
    return out.reshape(BATCH, MAXLEN, EMBED_DIM)


# speedup vs baseline: 6.3349x; 4.6369x over previous
import jax
import jax.numpy as jnp
from jax.experimental import pallas as pl

BATCH = 4096
MAXLEN = 200
EMBED_DIM = 64

SB = 8


def _add_kernel(x_ref, pos_ref, o_ref):
    o_ref[...] = x_ref[...] + pos_ref[...][:, :, None]


def kernel(x, pos_table):
    xt = x.transpose(1, 2, 0)
    out_t = pl.pallas_call(
        _add_kernel,
        grid=(MAXLEN // SB,),
        in_specs=[
            pl.BlockSpec((SB, EMBED_DIM, BATCH), lambda i: (i, 0, 0)),
            pl.BlockSpec((SB, EMBED_DIM), lambda i: (i, 0)),
        ],
        out_specs=pl.BlockSpec((SB, EMBED_DIM, BATCH), lambda i: (i, 0, 0)),
        out_shape=jax.ShapeDtypeStruct((MAXLEN, EMBED_DIM, BATCH), jnp.float32),
    )(xt, pos_table)
    return out_t.transpose(2, 0, 1)
